# baseline (speedup 1.0000x reference)
import jax
import jax.numpy as jnp
from jax.experimental import pallas as pl
from jax.experimental.pallas import tpu as pltpu

D_MODEL_ = 2048
N_EXP_ = 64
BM_ = 1024


def _router_body(xa_ref, xb_ref, w_ref, b_ref, o_ref):
    w = w_ref[...]
    bias = b_ref[...].reshape(N_EXP_, 1)
    dn = (((1,), (1,)), ((), ()))
    acc_a = jax.lax.dot_general(w, xa_ref[0], dn, preferred_element_type=jnp.float32)
    o_ref[0, :, :BM_] = acc_a + bias
    acc_b = jax.lax.dot_general(w, xb_ref[0], dn, preferred_element_type=jnp.float32)
    o_ref[0, :, BM_:] = acc_b + bias


def kernel(x, W, b):
    bsz, seq, d = x.shape
    grid = (bsz, seq // (2 * BM_))
    out = pl.pallas_call(
        _router_body,
        grid=grid,
        in_specs=[
            pl.BlockSpec((1, BM_, d), lambda i, j: (i, 2 * j, 0)),
            pl.BlockSpec((1, BM_, d), lambda i, j: (i, 2 * j + 1, 0)),
            pl.BlockSpec((N_EXP_, d), lambda i, j: (0, 0)),
            pl.BlockSpec((N_EXP_,), lambda i, j: (0,)),
        ],
        out_specs=pl.BlockSpec((1, N_EXP_, 2 * BM_), lambda i, j: (i, 0, j)),
        out_shape=jax.ShapeDtypeStruct((bsz, N_EXP_, seq), jnp.float32),
        compiler_params=pltpu.CompilerParams(
            dimension_semantics=("arbitrary", "arbitrary"),
        ),
    )(x, x, W, b)
    return jnp.swapaxes(out, 1, 2)
